# baseline (speedup 1.0000x reference)
import functools

import jax
import jax.numpy as jnp
from jax.experimental import pallas as pl

_NUM_EMBEDDINGS = 1024
_EMBEDDING_DIM = 64
_BETA = 0.25
_ROWS = 8 * 1024
_BLOCK_ROWS = 1024
_GRID = _ROWS // _BLOCK_ROWS


def _vq_block(x_ref, e_ref, q_ref, idx_ref, loss_ref):
    x = x_ref[...]
    e = e_ref[...]
    scores = jax.lax.dot_general(
        x, e, (((1,), (0,)), ((), ())), preferred_element_type=jnp.float32)
    rowsq = jnp.sum(x ** 2, axis=1, keepdims=True)
    esq = jnp.sum(e ** 2, axis=0, keepdims=True)
    d = rowsq - 2.0 * scores + esq
    idx = jnp.argmax(-d, axis=1)
    onehot = (jax.lax.broadcasted_iota(jnp.int32, (_BLOCK_ROWS, _NUM_EMBEDDINGS), 1)
              == idx[:, None]).astype(jnp.float32)
    q = jax.lax.dot_general(
        onehot, e, (((1,), (1,)), ((), ())), preferred_element_type=jnp.float32)
    q_ref[...] = x + (q - x)
    idx_ref[...] = idx.reshape(1, 1, _BLOCK_ROWS)
    diff = q - x
    part = jnp.sum(diff * diff).reshape(1, 1)

    @pl.when(pl.program_id(0) == 0)
    def _init():
        loss_ref[...] = jnp.zeros((1, 1), jnp.float32)

    loss_ref[...] += part


@functools.partial(jax.jit, static_argnames=())
def kernel(inputs, embeddings):
    input_shape = inputs.shape
    flat = inputs.reshape(_ROWS, _EMBEDDING_DIM)
    q_flat, idx3, loss_sum = pl.pallas_call(
        _vq_block,
        grid=(_GRID,),
        in_specs=[
            pl.BlockSpec((_BLOCK_ROWS, _EMBEDDING_DIM), lambda i: (i, 0)),
            pl.BlockSpec((_EMBEDDING_DIM, _NUM_EMBEDDINGS), lambda i: (0, 0)),
        ],
        out_specs=[
            pl.BlockSpec((_BLOCK_ROWS, _EMBEDDING_DIM), lambda i: (i, 0)),
            pl.BlockSpec((1, 1, _BLOCK_ROWS), lambda i: (i, 0, 0)),
            pl.BlockSpec((1, 1), lambda i: (0, 0)),
        ],
        out_shape=[
            jax.ShapeDtypeStruct((_ROWS, _EMBEDDING_DIM), jnp.float32),
            jax.ShapeDtypeStruct((_GRID, 1, _BLOCK_ROWS), jnp.int32),
            jax.ShapeDtypeStruct((1, 1), jnp.float32),
        ],
    )(flat, embeddings)
    quantized = q_flat.reshape(input_shape)
    commitment_loss = _BETA * (loss_sum[0, 0] / (_ROWS * _EMBEDDING_DIM))
    encoding_indices = idx3.reshape(_ROWS)
    return (quantized, commitment_loss, encoding_indices)


# speedup vs baseline: 1.0367x; 1.0367x over previous
import functools

import jax
import jax.numpy as jnp
from jax.experimental import pallas as pl

_NUM_EMBEDDINGS = 1024
_EMBEDDING_DIM = 64
_BETA = 0.25
_ROWS = 8 * 1024
_BLOCK_ROWS = 1024
_GRID = _ROWS // _BLOCK_ROWS


def _vq_block(x_ref, e_ref, q_ref, idx_ref, loss_ref):
    x = x_ref[...]
    e = e_ref[...]
    scores = jax.lax.dot_general(
        x, e, (((1,), (0,)), ((), ())), preferred_element_type=jnp.float32)
    rowsq = jnp.sum(x ** 2, axis=1, keepdims=True)
    esq = jnp.sum(e ** 2, axis=0, keepdims=True)
    d = rowsq - 2.0 * scores + esq
    idx = jnp.argmin(d, axis=1)
    onehot = (jax.lax.broadcasted_iota(jnp.int32, (_BLOCK_ROWS, _NUM_EMBEDDINGS), 1)
              == idx[:, None]).astype(jnp.float32)
    q = jax.lax.dot_general(
        onehot, e, (((1,), (1,)), ((), ())), preferred_element_type=jnp.float32)
    q_ref[...] = x + (q - x)
    idx_ref[...] = idx.reshape(1, 1, _BLOCK_ROWS)
    diff = q - x
    part = jnp.sum(diff * diff).reshape(1, 1)

    @pl.when(pl.program_id(0) == 0)
    def _init():
        loss_ref[...] = jnp.zeros((1, 1), jnp.float32)

    loss_ref[...] += part


@functools.partial(jax.jit, static_argnames=())
def kernel(inputs, embeddings):
    input_shape = inputs.shape
    flat = inputs.reshape(_ROWS, _EMBEDDING_DIM)
    q_flat, idx3, loss_sum = pl.pallas_call(
        _vq_block,
        grid=(_GRID,),
        in_specs=[
            pl.BlockSpec((_BLOCK_ROWS, _EMBEDDING_DIM), lambda i: (i, 0)),
            pl.BlockSpec((_EMBEDDING_DIM, _NUM_EMBEDDINGS), lambda i: (0, 0)),
        ],
        out_specs=[
            pl.BlockSpec((_BLOCK_ROWS, _EMBEDDING_DIM), lambda i: (i, 0)),
            pl.BlockSpec((1, 1, _BLOCK_ROWS), lambda i: (i, 0, 0)),
            pl.BlockSpec((1, 1), lambda i: (0, 0)),
        ],
        out_shape=[
            jax.ShapeDtypeStruct((_ROWS, _EMBEDDING_DIM), jnp.float32),
            jax.ShapeDtypeStruct((_GRID, 1, _BLOCK_ROWS), jnp.int32),
            jax.ShapeDtypeStruct((1, 1), jnp.float32),
        ],
    )(flat, embeddings)
    quantized = q_flat.reshape(input_shape)
    commitment_loss = _BETA * (loss_sum[0, 0] / (_ROWS * _EMBEDDING_DIM))
    encoding_indices = idx3.reshape(_ROWS)
    return (quantized, commitment_loss, encoding_indices)


# speedup vs baseline: 1.1373x; 1.0971x over previous
import functools

import jax
import jax.numpy as jnp
from jax.experimental import pallas as pl

_NUM_EMBEDDINGS = 1024
_EMBEDDING_DIM = 64
_BETA = 0.25
_ROWS = 8 * 1024
_BLOCK_ROWS = 2048
_GRID = _ROWS // _BLOCK_ROWS


def _vq_block(x_ref, e_ref, q_ref, idx_ref, loss_ref):
    x = x_ref[...]
    e = e_ref[...]
    scores = jax.lax.dot_general(
        x, e, (((1,), (0,)), ((), ())), preferred_element_type=jnp.float32)
    rowsq = jnp.sum(x ** 2, axis=1, keepdims=True)
    esq = jnp.sum(e ** 2, axis=0, keepdims=True)
    d = rowsq - 2.0 * scores + esq
    idx = jnp.argmin(d, axis=1)
    onehot = (jax.lax.broadcasted_iota(jnp.int32, (_BLOCK_ROWS, _NUM_EMBEDDINGS), 1)
              == idx[:, None]).astype(jnp.float32)
    q = jax.lax.dot_general(
        onehot, e, (((1,), (1,)), ((), ())), preferred_element_type=jnp.float32)
    q_ref[...] = x + (q - x)
    idx_ref[...] = idx.reshape(1, 1, _BLOCK_ROWS)
    diff = q - x
    part = jnp.sum(diff * diff).reshape(1, 1)

    @pl.when(pl.program_id(0) == 0)
    def _init():
        loss_ref[...] = jnp.zeros((1, 1), jnp.float32)

    loss_ref[...] += part


@functools.partial(jax.jit, static_argnames=())
def kernel(inputs, embeddings):
    input_shape = inputs.shape
    flat = inputs.reshape(_ROWS, _EMBEDDING_DIM)
    q_flat, idx3, loss_sum = pl.pallas_call(
        _vq_block,
        grid=(_GRID,),
        in_specs=[
            pl.BlockSpec((_BLOCK_ROWS, _EMBEDDING_DIM), lambda i: (i, 0)),
            pl.BlockSpec((_EMBEDDING_DIM, _NUM_EMBEDDINGS), lambda i: (0, 0)),
        ],
        out_specs=[
            pl.BlockSpec((_BLOCK_ROWS, _EMBEDDING_DIM), lambda i: (i, 0)),
            pl.BlockSpec((1, 1, _BLOCK_ROWS), lambda i: (i, 0, 0)),
            pl.BlockSpec((1, 1), lambda i: (0, 0)),
        ],
        out_shape=[
            jax.ShapeDtypeStruct((_ROWS, _EMBEDDING_DIM), jnp.float32),
            jax.ShapeDtypeStruct((_GRID, 1, _BLOCK_ROWS), jnp.int32),
            jax.ShapeDtypeStruct((1, 1), jnp.float32),
        ],
    )(flat, embeddings)
    quantized = q_flat.reshape(input_shape)
    commitment_loss = _BETA * (loss_sum[0, 0] / (_ROWS * _EMBEDDING_DIM))
    encoding_indices = idx3.reshape(_ROWS)
    return (quantized, commitment_loss, encoding_indices)


# speedup vs baseline: 1.1544x; 1.0151x over previous
import functools

import jax
import jax.numpy as jnp
from jax.experimental import pallas as pl

_NUM_EMBEDDINGS = 1024
_EMBEDDING_DIM = 64
_BETA = 0.25
_ROWS = 8 * 1024
_BLOCK_ROWS = 4096
_GRID = _ROWS // _BLOCK_ROWS


def _vq_block(x_ref, e_ref, q_ref, idx_ref, loss_ref):
    x = x_ref[...]
    e = e_ref[...]
    scores = jax.lax.dot_general(
        x, e, (((1,), (0,)), ((), ())), preferred_element_type=jnp.float32)
    rowsq = jnp.sum(x ** 2, axis=1, keepdims=True)
    esq = jnp.sum(e ** 2, axis=0, keepdims=True)
    d = rowsq - 2.0 * scores + esq
    idx = jnp.argmin(d, axis=1)
    onehot = (jax.lax.broadcasted_iota(jnp.int32, (_BLOCK_ROWS, _NUM_EMBEDDINGS), 1)
              == idx[:, None]).astype(jnp.float32)
    q = jax.lax.dot_general(
        onehot, e, (((1,), (1,)), ((), ())), preferred_element_type=jnp.float32)
    q_ref[...] = x + (q - x)
    idx_ref[...] = idx.reshape(1, 1, _BLOCK_ROWS)
    diff = q - x
    part = jnp.sum(diff * diff).reshape(1, 1)

    @pl.when(pl.program_id(0) == 0)
    def _init():
        loss_ref[...] = jnp.zeros((1, 1), jnp.float32)

    loss_ref[...] += part


@functools.partial(jax.jit, static_argnames=())
def kernel(inputs, embeddings):
    input_shape = inputs.shape
    flat = inputs.reshape(_ROWS, _EMBEDDING_DIM)
    q_flat, idx3, loss_sum = pl.pallas_call(
        _vq_block,
        grid=(_GRID,),
        in_specs=[
            pl.BlockSpec((_BLOCK_ROWS, _EMBEDDING_DIM), lambda i: (i, 0)),
            pl.BlockSpec((_EMBEDDING_DIM, _NUM_EMBEDDINGS), lambda i: (0, 0)),
        ],
        out_specs=[
            pl.BlockSpec((_BLOCK_ROWS, _EMBEDDING_DIM), lambda i: (i, 0)),
            pl.BlockSpec((1, 1, _BLOCK_ROWS), lambda i: (i, 0, 0)),
            pl.BlockSpec((1, 1), lambda i: (0, 0)),
        ],
        out_shape=[
            jax.ShapeDtypeStruct((_ROWS, _EMBEDDING_DIM), jnp.float32),
            jax.ShapeDtypeStruct((_GRID, 1, _BLOCK_ROWS), jnp.int32),
            jax.ShapeDtypeStruct((1, 1), jnp.float32),
        ],
    )(flat, embeddings)
    quantized = q_flat.reshape(input_shape)
    commitment_loss = _BETA * (loss_sum[0, 0] / (_ROWS * _EMBEDDING_DIM))
    encoding_indices = idx3.reshape(_ROWS)
    return (quantized, commitment_loss, encoding_indices)
